# baseline (speedup 1.0000x reference)
import functools

import jax
import jax.numpy as jnp
from jax import lax
from jax.experimental import pallas as pl
from jax.experimental.pallas import tpu as pltpu
from jax.experimental.pallas import tpu_sc as plsc

N = 10000
NP = 10240
E = 160000
D0, D1, D2, K = 256, 128, 32, 10
NC, NS, L = 2, 16, 16
NW = NC * NS
EW = 5120
C = 64
CH = EW // C
EP = NW * EW
RB = 128
GRID = NP // RB
RB1 = 200
GRID1 = N // RB1
SEG = NP // NS

_SCPARAMS = dict(needs_layout_passes=False)


@functools.cache
def _mesh():
    return plsc.VectorSubcoreMesh(
        core_axis_name="c", subcore_axis_name="s",
        num_cores=NC, num_subcores=NS)


def _tc1_body(f_ref, w_ref, asrc_ref, adst_ref, h_ref, s_ref, d_ref):
    h = jnp.dot(f_ref[...], w_ref[...], preferred_element_type=jnp.float32)
    h_ref[...] = h
    s_ref[...] = jnp.sum(h * asrc_ref[...], axis=1)[None, None, :]
    d_ref[...] = jnp.sum(h * adst_ref[...], axis=1)[None, None, :]


def _tc1(features, W_enc, asrc2, adst2):
    return pl.pallas_call(
        _tc1_body,
        grid=(GRID1,),
        in_specs=[pl.BlockSpec((RB1, D0), lambda i: (i, 0)),
                  pl.BlockSpec((D0, D1), lambda i: (0, 0)),
                  pl.BlockSpec((1, D1), lambda i: (0, 0)),
                  pl.BlockSpec((1, D1), lambda i: (0, 0))],
        out_specs=[pl.BlockSpec((RB1, D1), lambda i: (i, 0)),
                   pl.BlockSpec((1, 1, RB1), lambda i: (i, 0, 0)),
                   pl.BlockSpec((1, 1, RB1), lambda i: (i, 0, 0))],
        out_shape=[jax.ShapeDtypeStruct((N, D1), jnp.float32),
                   jax.ShapeDtypeStruct((GRID1, 1, RB1), jnp.float32),
                   jax.ShapeDtypeStruct((GRID1, 1, RB1), jnp.float32)],
    )(features, W_enc, asrc2, adst2)


@functools.cache
def _scb_kernel():
    return pl.kernel(
        _scb_body,
        out_type=[jax.ShapeDtypeStruct((EP,), jnp.float32),
                  jax.ShapeDtypeStruct((NC, NP), jnp.float32),
                  jax.ShapeDtypeStruct((NC, NP, D1), jnp.float32)],
        mesh=_mesh(),
        compiler_params=pltpu.CompilerParams(**_SCPARAMS),
        scratch_types=[pltpu.VMEM((N,), jnp.float32),
                       pltpu.VMEM((N,), jnp.float32)]
                      + [pltpu.VMEM((C,), jnp.int32)] * 4
                      + [pltpu.VMEM((C,), jnp.float32)] * 2
                      + [pltpu.VMEM((C, D1), jnp.float32)] * 2
                      + [pltpu.SemaphoreType.DMA] * 6
                      + [pltpu.VMEM_SHARED((NP,), jnp.float32),
                         pltpu.VMEM_SHARED((NP, D1), jnp.float32)])


def _scb_body(src_hbm, dst_hbm, s_hbm, d_hbm, z1_hbm, z2_hbm, h_hbm,
              e_hbm, denp_hbm, encp_hbm,
              s_v, d_v, srci0, srci1, dsti0, dsti1,
              e0, e1, rows0, rows1,
              sem_l0, sem_l1, sem_g0, sem_g1, sem_o0, sem_o1,
              den_sp, acc_sp):
    cid = lax.axis_index("c")
    sid = lax.axis_index("s")
    wid = sid * NC + cid
    srci = (srci0, srci1)
    dsti = (dsti0, dsti1)
    ev = (e0, e1)
    rows = (rows0, rows1)
    sem_l = (sem_l0, sem_l1)
    sem_g = (sem_g0, sem_g1)
    sem_o = (sem_o0, sem_o1)
    pltpu.sync_copy(s_hbm, s_v)
    pltpu.sync_copy(d_hbm, d_v)
    pltpu.sync_copy(z1_hbm, den_sp.at[pl.ds(sid * SEG, SEG)])
    pltpu.sync_copy(z2_hbm, acc_sp.at[pl.ds(sid * SEG, SEG), :])
    plsc.subcore_barrier()
    iota = lax.broadcasted_iota(jnp.int32, (L,), 0)

    def round_(t, carry):
        base0 = wid * EW + 2 * t * C
        bases = (base0, base0 + C)
        gs = []
        for b in range(2):
            pltpu.sync_copy(src_hbm.at[pl.ds(bases[b], C)], srci[b])
            pltpu.sync_copy(dst_hbm.at[pl.ds(bases[b], C)], dsti[b])
            gs.append(pltpu.async_copy(h_hbm.at[srci[b]], rows[b], sem_g[b]))
        for b in range(2):
            for g in range(C // L):
                si = srci[b][pl.ds(g * L, L)]
                di = dsti[b][pl.ds(g * L, L)]
                lg = plsc.load_gather(s_v, [si]) + plsc.load_gather(d_v, [di])
                lg = jnp.where(lg >= 0, lg, 0.2 * lg)
                e = jnp.exp(lg)
                eid = bases[b] + g * L + iota
                ev[b][pl.ds(g * L, L)] = jnp.where(eid < E, e, 0.0)
            pltpu.sync_copy(ev[b], e_hbm.at[pl.ds(bases[b], C)])
            pltpu.sync_copy(ev[b], den_sp.at[dsti[b]], add=True)
        for b in range(2):
            gs[b].wait()
            rb_ref = rows[b]
            ab_ref = ev[b]

            def scale(i, carry2, rb_ref=rb_ref, ab_ref=ab_ref):
                ab = plsc.load_gather(ab_ref, [jnp.zeros((L,), jnp.int32) + i])
                for k in range(D1 // L):
                    rb_ref[i, pl.ds(k * L, L)] = rb_ref[i, pl.ds(k * L, L)] * ab
                return carry2

            lax.fori_loop(0, C, scale, 0, unroll=2)
            pltpu.sync_copy(rows[b], acc_sp.at[dsti[b]], add=True)
        return carry

    lax.fori_loop(0, CH // 2, round_, 0)
    plsc.subcore_barrier()
    pltpu.sync_copy(den_sp.at[pl.ds(sid * SEG, SEG)],
                    denp_hbm.at[cid, pl.ds(sid * SEG, SEG)])
    pltpu.sync_copy(acc_sp.at[pl.ds(sid * SEG, SEG), :],
                    encp_hbm.at[cid, pl.ds(sid * SEG, SEG), :])


@functools.cache
def _scc_kernel():
    return pl.kernel(
        _scc_body,
        out_type=[jax.ShapeDtypeStruct((NC, NP, D2), jnp.float32)],
        mesh=_mesh(),
        compiler_params=pltpu.CompilerParams(use_tc_tiling_on_sc=False,
                                             **_SCPARAMS),
        scratch_types=[pltpu.VMEM((C,), jnp.int32)] * 4
                      + [pltpu.VMEM((C,), jnp.float32)] * 2
                      + [pltpu.VMEM((C, D2), jnp.float32)] * 2
                      + [pltpu.SemaphoreType.DMA] * 6
                      + [pltpu.VMEM_SHARED((NP, D2), jnp.float32)])


def _scc_body(src_hbm, dst_hbm, e_hbm, lat_hbm, z3_hbm,
              aggp_hbm,
              srci0, srci1, dsti0, dsti1, al0, al1, rows0, rows1,
              sem_l0, sem_l1, sem_g0, sem_g1, sem_o0, sem_o1, acc_sp):
    cid = lax.axis_index("c")
    sid = lax.axis_index("s")
    wid = sid * NC + cid
    srci = (srci0, srci1)
    dsti = (dsti0, dsti1)
    al = (al0, al1)
    rows = (rows0, rows1)
    sem_l = (sem_l0, sem_l1)
    sem_g = (sem_g0, sem_g1)
    sem_o = (sem_o0, sem_o1)
    pltpu.sync_copy(z3_hbm, acc_sp.at[pl.ds(sid * SEG, SEG), :])
    plsc.subcore_barrier()

    def round_(t, carry):
        base0 = wid * EW + 2 * t * C
        bases = (base0, base0 + C)
        gs = []
        for b in range(2):
            pltpu.sync_copy(src_hbm.at[pl.ds(bases[b], C)], srci[b])
            pltpu.sync_copy(dst_hbm.at[pl.ds(bases[b], C)], dsti[b])
            pltpu.sync_copy(e_hbm.at[pl.ds(bases[b], C)], al[b])
            gs.append(pltpu.async_copy(lat_hbm.at[srci[b]], rows[b], sem_g[b]))
        for b in range(2):
            gs[b].wait()
            rb_ref = rows[b]
            ab_ref = al[b]

            def scale(i, carry2, rb_ref=rb_ref, ab_ref=ab_ref):
                ab = plsc.load_gather(ab_ref, [jnp.zeros((L,), jnp.int32) + i])
                for k in range(D2 // L):
                    rb_ref[i, pl.ds(k * L, L)] = rb_ref[i, pl.ds(k * L, L)] * ab
                return carry2

            lax.fori_loop(0, C, scale, 0, unroll=4)
            pltpu.sync_copy(rows[b], acc_sp.at[dsti[b]], add=True)
        return carry

    lax.fori_loop(0, CH // 2, round_, 0)
    plsc.subcore_barrier()
    pltpu.sync_copy(acc_sp.at[pl.ds(sid * SEG, SEG), :],
                    aggp_hbm.at[cid, pl.ds(sid * SEG, SEG), :])


def _tc2_body(encp_ref, dp0_ref, dp1_ref, w1_ref, lat_ref):
    den = dp0_ref[...] + dp1_ref[...] + 1e-16
    enc = (encp_ref[0] + encp_ref[1]) / den
    enc = jnp.where(enc > 0, enc, jnp.exp(jnp.minimum(enc, 0.0)) - 1.0)
    lat_ref[...] = jnp.dot(enc, w1_ref[...], preferred_element_type=jnp.float32)


def _tc2(encp, dp0, dp1, W1):
    return pl.pallas_call(
        _tc2_body,
        grid=(GRID,),
        in_specs=[pl.BlockSpec((NC, RB, D1), lambda i: (0, i, 0)),
                  pl.BlockSpec((RB, 1), lambda i: (i, 0)),
                  pl.BlockSpec((RB, 1), lambda i: (i, 0)),
                  pl.BlockSpec((D1, D2), lambda i: (0, 0))],
        out_specs=pl.BlockSpec((RB, D2), lambda i: (i, 0)),
        out_shape=jax.ShapeDtypeStruct((NP, D2), jnp.float32),
    )(encp, dp0, dp1, W1)


def _tcq_body(lat_ref, cent_ref, q_ref):
    lat = lat_ref[...]
    cent = cent_ref[...]
    gmat = lax.dot_general(lat, cent, (((1,), (1,)), ((), ())),
                           preferred_element_type=jnp.float32)
    l2 = jnp.sum(lat * lat, axis=1, keepdims=True)
    c2 = jnp.sum(cent * cent, axis=1)[None, :]
    d2 = l2 - 2.0 * gmat + c2
    qu = 1.0 / (1.0 + d2 + 1e-6)
    q_ref[...] = qu / jnp.sum(qu, axis=1, keepdims=True)


def _tcq(latent, centroids):
    return pl.pallas_call(
        _tcq_body,
        grid=(GRID,),
        in_specs=[pl.BlockSpec((RB, D2), lambda i: (i, 0)),
                  pl.BlockSpec((K, D2), lambda i: (0, 0))],
        out_specs=pl.BlockSpec((RB, K), lambda i: (i, 0)),
        out_shape=jax.ShapeDtypeStruct((NP, K), jnp.float32),
    )(latent, centroids)


def _tc3_body(ap_ref, dp0_ref, dp1_ref, w1_ref, wenc_ref, recon_ref):
    den = dp0_ref[...] + dp1_ref[...] + 1e-16
    agg = (ap_ref[0] + ap_ref[1]) / den
    dec = lax.dot_general(agg, w1_ref[...], (((1,), (1,)), ((), ())),
                          preferred_element_type=jnp.float32)
    dec = jnp.where(dec > 0, dec, jnp.exp(jnp.minimum(dec, 0.0)) - 1.0)
    recon_ref[...] = lax.dot_general(dec, wenc_ref[...], (((1,), (1,)), ((), ())),
                                     preferred_element_type=jnp.float32)


def _tc3(aggp, dp0, dp1, W1, W_enc):
    return pl.pallas_call(
        _tc3_body,
        grid=(GRID,),
        in_specs=[pl.BlockSpec((NC, RB, D2), lambda i: (0, i, 0)),
                  pl.BlockSpec((RB, 1), lambda i: (i, 0)),
                  pl.BlockSpec((RB, 1), lambda i: (i, 0)),
                  pl.BlockSpec((D1, D2), lambda i: (0, 0)),
                  pl.BlockSpec((D0, D1), lambda i: (0, 0))],
        out_specs=pl.BlockSpec((RB, D0), lambda i: (i, 0)),
        out_shape=jax.ShapeDtypeStruct((NP, D0), jnp.float32),
    )(aggp, dp0, dp1, W1, W_enc)


def kernel(features, edge_index, W_enc, att_src, att_dst, W1, centroids):
    src = edge_index[0].astype(jnp.int32)
    dst = edge_index[1].astype(jnp.int32)
    pad = EP - E
    src_p = jnp.concatenate([src, jnp.zeros((pad,), jnp.int32)])
    dst_p = jnp.concatenate([dst, jnp.arange(pad, dtype=jnp.int32)])
    z1 = jnp.zeros((SEG,), jnp.float32)
    z2 = jnp.zeros((SEG, D1), jnp.float32)
    z3 = jnp.zeros((SEG, D2), jnp.float32)
    asrc2 = att_src.reshape(1, D1)
    adst2 = att_dst.reshape(1, D1)

    h, s2, d2m = _tc1(features, W_enc, asrc2, adst2)
    s = s2.reshape(N)
    dv = d2m.reshape(N)
    e_all, denp, encp = _scb_kernel()(src_p, dst_p, s, dv, z1, z2, h)
    dp0 = denp[0].reshape(NP, 1)
    dp1 = denp[1].reshape(NP, 1)
    latent = _tc2(encp, dp0, dp1, W1)
    q = _tcq(latent, centroids)
    (aggp,) = _scc_kernel()(src_p, dst_p, e_all, latent, z3)
    recon = _tc3(aggp, dp0, dp1, W1, W_enc)
    return latent[:N], recon[:N], q[:N]
